# baseline (device time: 19391 ns/iter reference)
import jax
import jax.numpy as jnp
from jax import lax
from jax.experimental import pallas as pl
from jax.experimental.pallas import tpu as pltpu

N_DEV = 32
NG = 8
NC = 4
N_CHUNKS = 4
SLAB = 128


def kernel(x, w_mat):
    m_per, k = x.shape
    n = w_mat.shape[1]
    n_per = n // N_DEV
    n_chunk = n // N_CHUNKS
    slabs_per_chunk = n_chunk // SLAB

    def body(x_ref, w_ref, out_ref, y_ref, p1_ref, pk_ref, p2_ref,
             send1, recv1, send2, recv2, ack1, ack2):
        my = lax.axis_index("i")
        my_g = my // NC
        my_c = lax.rem(my, NC)

        barrier_sem = pltpu.get_barrier_semaphore()
        pl.semaphore_signal(
            barrier_sem, inc=1, device_id=(my,),
            device_id_type=pl.DeviceIdType.MESH,
        )
        pl.semaphore_wait(barrier_sem, 1)

        x_val = x_ref[...]

        for kk in range(N_CHUNKS):
            yc = jnp.dot(
                x_val, w_ref[:, kk * n_chunk:(kk + 1) * n_chunk],
                preferred_element_type=jnp.float32,
            )
            yc = yc * jax.nn.sigmoid(yc)
            y_ref[:, kk * n_chunk:(kk + 1) * n_chunk] = yc

            for s in range(slabs_per_chunk):
                gp = kk * slabs_per_chunk + s
                rdma = pltpu.make_async_remote_copy(
                    src_ref=y_ref.at[:, pl.ds(gp * SLAB, SLAB)],
                    dst_ref=p1_ref.at[my_g],
                    send_sem=send1.at[gp],
                    recv_sem=recv1.at[my_g],
                    device_id=(gp * NC + my_c,),
                    device_id_type=pl.DeviceIdType.MESH,
                )
                rdma.start()

        for g2 in range(NG):
            recv = pltpu.make_async_remote_copy(
                src_ref=y_ref.at[:, pl.ds(0, SLAB)],
                dst_ref=p1_ref.at[g2],
                send_sem=send1.at[0],
                recv_sem=recv1.at[g2],
                device_id=(my,),
                device_id_type=pl.DeviceIdType.MESH,
            )
            recv.wait_recv()
            slab = p1_ref[g2]
            for cp in range(NC):
                pk_ref[cp, g2, :, :] = slab[:, cp * n_per:(cp + 1) * n_per]

        for g2 in range(NG):
            pl.semaphore_signal(
                ack1, inc=1, device_id=(g2 * NC + my_c,),
                device_id_type=pl.DeviceIdType.MESH,
            )

        for cp in range(NC):
            rdma = pltpu.make_async_remote_copy(
                src_ref=pk_ref.at[cp],
                dst_ref=p2_ref.at[my_c],
                send_sem=send2.at[cp],
                recv_sem=recv2.at[my_c],
                device_id=(my_g * NC + cp,),
                device_id_type=pl.DeviceIdType.MESH,
            )
            rdma.start()

        for c2 in range(NC):
            recv = pltpu.make_async_remote_copy(
                src_ref=pk_ref.at[0],
                dst_ref=p2_ref.at[c2],
                send_sem=send2.at[0],
                recv_sem=recv2.at[c2],
                device_id=(my,),
                device_id_type=pl.DeviceIdType.MESH,
            )
            recv.wait_recv()
            for g2 in range(NG):
                src_dev = g2 * NC + c2
                out_ref[src_dev * m_per:(src_dev + 1) * m_per, :] = (
                    p2_ref[c2, g2]
                )

        for c2 in range(NC):
            pl.semaphore_signal(
                ack2, inc=1, device_id=(my_g * NC + c2,),
                device_id_type=pl.DeviceIdType.MESH,
            )

        for gp in range(NG):
            sem = pltpu.make_async_remote_copy(
                src_ref=y_ref.at[:, pl.ds(0, SLAB)],
                dst_ref=p1_ref.at[0],
                send_sem=send1.at[gp],
                recv_sem=recv1.at[0],
                device_id=(my,),
                device_id_type=pl.DeviceIdType.MESH,
            )
            sem.wait_send()
        for cp in range(NC):
            sem = pltpu.make_async_remote_copy(
                src_ref=pk_ref.at[0],
                dst_ref=p2_ref.at[0],
                send_sem=send2.at[cp],
                recv_sem=recv2.at[0],
                device_id=(my,),
                device_id_type=pl.DeviceIdType.MESH,
            )
            sem.wait_send()
        pl.semaphore_wait(ack1, NG)
        pl.semaphore_wait(ack2, NC)

    return pl.pallas_call(
        body,
        out_shape=jax.ShapeDtypeStruct((N_DEV * m_per, n_per), jnp.float32),
        in_specs=[
            pl.BlockSpec(memory_space=pltpu.VMEM),
            pl.BlockSpec(memory_space=pltpu.VMEM),
        ],
        out_specs=pl.BlockSpec(memory_space=pltpu.VMEM),
        compiler_params=pltpu.CompilerParams(collective_id=0),
        scratch_shapes=[
            pltpu.VMEM((m_per, n), jnp.float32),
            pltpu.VMEM((NG, m_per, SLAB), jnp.float32),
            pltpu.VMEM((NC, NG, m_per, n_per), jnp.float32),
            pltpu.VMEM((NC, NG, m_per, n_per), jnp.float32),
            pltpu.SemaphoreType.DMA((NG,)),
            pltpu.SemaphoreType.DMA((NG,)),
            pltpu.SemaphoreType.DMA((NC,)),
            pltpu.SemaphoreType.DMA((NC,)),
            pltpu.SemaphoreType.REGULAR,
            pltpu.SemaphoreType.REGULAR,
        ],
    )(x, w_mat)


# device time: 16737 ns/iter; 1.1586x vs baseline; 1.1586x over previous
import jax
import jax.numpy as jnp
from jax import lax
from jax.experimental import pallas as pl
from jax.experimental.pallas import tpu as pltpu

N_DEV = 32
NG = 8
NC = 4
N_CHUNKS = 4
SLAB = 128


def kernel(x, w_mat):
    m_per, k = x.shape
    n = w_mat.shape[1]
    n_per = n // N_DEV
    n_chunk = n // N_CHUNKS
    slabs_per_chunk = n_chunk // SLAB

    def body(x_ref, w_ref, out_ref, y_ref, p1_ref, pk_ref, p2_ref,
             send1, recv1, send2, recv2, ack1, ack2):
        my = lax.axis_index("i")
        my_g = my // NC
        my_c = lax.rem(my, NC)

        barrier_sem = pltpu.get_barrier_semaphore()
        pl.semaphore_signal(
            barrier_sem, inc=1, device_id=(my,),
            device_id_type=pl.DeviceIdType.MESH,
        )
        pl.semaphore_wait(barrier_sem, 1)

        x_val = x_ref[...]

        for kk in range(N_CHUNKS):
            yc = jnp.dot(
                x_val, w_ref[:, kk * n_chunk:(kk + 1) * n_chunk],
                preferred_element_type=jnp.float32,
            )
            yc = yc * jax.nn.sigmoid(yc)
            y_ref[:, kk * n_chunk:(kk + 1) * n_chunk] = yc

            for s in range(slabs_per_chunk):
                gp = kk * slabs_per_chunk + s
                rdma = pltpu.make_async_remote_copy(
                    src_ref=y_ref.at[:, pl.ds(gp * SLAB, SLAB)],
                    dst_ref=p1_ref.at[my_g],
                    send_sem=send1.at[gp],
                    recv_sem=recv1.at[my_g],
                    device_id=(gp * NC + my_c,),
                    device_id_type=pl.DeviceIdType.MESH,
                )
                rdma.start()

        for g2 in range(NG):
            recv = pltpu.make_async_remote_copy(
                src_ref=y_ref.at[:, pl.ds(0, SLAB)],
                dst_ref=p1_ref.at[g2],
                send_sem=send1.at[0],
                recv_sem=recv1.at[g2],
                device_id=(my,),
                device_id_type=pl.DeviceIdType.MESH,
            )
            recv.wait_recv()
            slab = p1_ref[g2]
            for cp in range(NC):
                pk_ref[cp, :, g2 * n_per:(g2 + 1) * n_per] = (
                    slab[:, cp * n_per:(cp + 1) * n_per]
                )

        for g2 in range(NG):
            pl.semaphore_signal(
                ack1, inc=1, device_id=(g2 * NC + my_c,),
                device_id_type=pl.DeviceIdType.MESH,
            )

        for cp in range(NC):
            rdma = pltpu.make_async_remote_copy(
                src_ref=pk_ref.at[cp],
                dst_ref=p2_ref.at[my_c],
                send_sem=send2.at[cp],
                recv_sem=recv2.at[my_c],
                device_id=(my_g * NC + cp,),
                device_id_type=pl.DeviceIdType.MESH,
            )
            rdma.start()

        for c2 in range(NC):
            recv = pltpu.make_async_remote_copy(
                src_ref=pk_ref.at[0],
                dst_ref=p2_ref.at[c2],
                send_sem=send2.at[0],
                recv_sem=recv2.at[c2],
                device_id=(my,),
                device_id_type=pl.DeviceIdType.MESH,
            )
            recv.wait_recv()
            stack = p2_ref[c2]
            for g2 in range(NG):
                src_dev = g2 * NC + c2
                out_ref[src_dev * m_per:(src_dev + 1) * m_per, :] = (
                    stack[:, g2 * n_per:(g2 + 1) * n_per]
                )

        for c2 in range(NC):
            pl.semaphore_signal(
                ack2, inc=1, device_id=(my_g * NC + c2,),
                device_id_type=pl.DeviceIdType.MESH,
            )

        for gp in range(NG):
            sem = pltpu.make_async_remote_copy(
                src_ref=y_ref.at[:, pl.ds(0, SLAB)],
                dst_ref=p1_ref.at[0],
                send_sem=send1.at[gp],
                recv_sem=recv1.at[0],
                device_id=(my,),
                device_id_type=pl.DeviceIdType.MESH,
            )
            sem.wait_send()
        for cp in range(NC):
            sem = pltpu.make_async_remote_copy(
                src_ref=pk_ref.at[0],
                dst_ref=p2_ref.at[0],
                send_sem=send2.at[cp],
                recv_sem=recv2.at[0],
                device_id=(my,),
                device_id_type=pl.DeviceIdType.MESH,
            )
            sem.wait_send()
        pl.semaphore_wait(ack1, NG)
        pl.semaphore_wait(ack2, NC)

    return pl.pallas_call(
        body,
        out_shape=jax.ShapeDtypeStruct((N_DEV * m_per, n_per), jnp.float32),
        in_specs=[
            pl.BlockSpec(memory_space=pltpu.VMEM),
            pl.BlockSpec(memory_space=pltpu.VMEM),
        ],
        out_specs=pl.BlockSpec(memory_space=pltpu.VMEM),
        compiler_params=pltpu.CompilerParams(collective_id=0),
        scratch_shapes=[
            pltpu.VMEM((m_per, n), jnp.float32),
            pltpu.VMEM((NG, m_per, SLAB), jnp.float32),
            pltpu.VMEM((NC, m_per, NG * n_per), jnp.float32),
            pltpu.VMEM((NC, m_per, NG * n_per), jnp.float32),
            pltpu.SemaphoreType.DMA((NG,)),
            pltpu.SemaphoreType.DMA((NG,)),
            pltpu.SemaphoreType.DMA((NC,)),
            pltpu.SemaphoreType.DMA((NC,)),
            pltpu.SemaphoreType.REGULAR,
            pltpu.SemaphoreType.REGULAR,
        ],
    )(x, w_mat)
